# baseline (speedup 1.0000x reference)
import functools

import jax
import jax.numpy as jnp
from jax import lax
from jax.experimental import pallas as pl
from jax.experimental.pallas import tpu as pltpu
from jax.experimental.pallas import tpu_sc as plsc


def _vq_idx_body(z_ref, cb_ref, idx_ref, cbn_ref):
    @pl.when(jnp.logical_and(pl.program_id(0) == 0, pl.program_id(1) == 0))
    def _():
        cb0 = cb_ref[...]
        cbn_ref[...] = 0.5 * jnp.sum(cb0 * cb0, axis=1, keepdims=True)

    zb = z_ref[0]
    cb = cb_ref[...]
    scores = jax.lax.dot_general(
        cb, zb, (((1,), (0,)), ((), ())),
        preferred_element_type=jnp.float32)
    rank = scores - cbn_ref[...]
    idx = jnp.argmax(rank, axis=0).astype(jnp.int32)
    idx_ref[0, 0, 0] = idx


def _tc_indices(z3, codebook, T):
    B, D, HW = z3.shape
    K = codebook.shape[0]
    NT = HW // T
    idx = pl.pallas_call(
        _vq_idx_body,
        grid=(B, NT),
        in_specs=[
            pl.BlockSpec((1, D, T), lambda b, t: (b, 0, t)),
            pl.BlockSpec((K, D), lambda b, t: (0, 0)),
        ],
        out_specs=pl.BlockSpec((1, 1, 1, T), lambda b, t: (b, t, 0, 0)),
        out_shape=jax.ShapeDtypeStruct((B, NT, 1, T), jnp.int32),
        scratch_shapes=[pltpu.VMEM((K, 1), jnp.float32)],
    )(z3, codebook)
    return idx.reshape(B, HW)


def _make_sc_gather(N, D, C, NB=3):
    info = plsc.get_sparse_core_info()
    NC, NS = info.num_cores, info.num_subcores
    NW = NC * NS
    n_per_w = N // NW
    nch = n_per_w // C
    mesh = plsc.VectorSubcoreMesh(core_axis_name="c", subcore_axis_name="s")

    @functools.partial(
        pl.kernel, mesh=mesh,
        out_type=jax.ShapeDtypeStruct((N, D), jnp.float32),
        scratch_types=[
            pltpu.VMEM((n_per_w,), jnp.int32),
        ] + [pltpu.VMEM((C, D), jnp.float32) for _ in range(NB)]
          + [pltpu.SemaphoreType.DMA((NB,)), pltpu.SemaphoreType.DMA((NB,))],
    )
    def gather(table_hbm, idx_hbm, out_hbm, idx_all, *bufs_and_sems):
        bufs = list(bufs_and_sems[:NB])
        gsem, ssem = bufs_and_sems[NB], bufs_and_sems[NB + 1]
        wid = lax.axis_index("s") * NC + lax.axis_index("c")
        base = wid * n_per_w
        pltpu.sync_copy(idx_hbm.at[pl.ds(base, n_per_w)], idx_all)

        def start_gather(i):
            b = i % NB
            return pltpu.async_copy(
                table_hbm.at[idx_all.at[pl.ds(i * C, C)]], bufs[b],
                gsem.at[b])

        def start_store(i):
            b = i % NB
            return pltpu.async_copy(
                bufs[b], out_hbm.at[pl.ds(base + i * C, C)], ssem.at[b])

        LAG = max(NB // 2, 1)
        ghandles = [None] * nch
        shandles = [None] * nch
        for i in range(nch + LAG):
            if i < nch:
                if i >= NB:
                    shandles[i - NB].wait()
                ghandles[i] = start_gather(i)
            k = i - LAG
            if 0 <= k < nch:
                ghandles[k].wait()
                shandles[k] = start_store(k)
        for k in range(max(nch - NB, 0), nch):
            shandles[k].wait()

    return gather


def kernel(z, codebook):
    B, D, H, W = z.shape
    HW = H * W
    z3 = z.reshape(B, D, HW)
    T = min(4096, HW)
    idx = _tc_indices(z3, codebook, T)
    N = B * HW
    x = _make_sc_gather(N, D, 64, NB=7)(codebook, idx.reshape(N))
    return x.reshape(B, HW, D), idx


# speedup vs baseline: 3.0373x; 3.0373x over previous
import jax
import jax.numpy as jnp
from jax.experimental import pallas as pl
from jax.experimental.pallas import tpu as pltpu

BB = 2


def _vq_body(z_ref, cb_ref, x_ref, idx_ref, cbn_ref):
    @pl.when(pl.program_id(0) == 0)
    def _():
        cb0 = cb_ref[...]
        cbn_ref[...] = 0.5 * jnp.sum(cb0 * cb0, axis=1, keepdims=True)

    cb = cb_ref[...]
    K = cb.shape[0]
    for j in range(BB):
        zb = z_ref[j]
        T = zb.shape[1]
        scores = jax.lax.dot_general(
            cb, zb, (((1,), (0,)), ((), ())),
            preferred_element_type=jnp.float32)
        rank = scores - cbn_ref[...]
        idx = jnp.argmax(rank, axis=0).astype(jnp.int32)
        onehot = (jax.lax.broadcasted_iota(jnp.int32, (K, T), 0)
                  == idx[None, :]).astype(jnp.float32)
        xv = jax.lax.dot_general(
            onehot, cb, (((0,), (0,)), ((), ())),
            preferred_element_type=jnp.float32)
        x_ref[j] = xv
        idx_ref[j, 0, 0] = idx


def kernel(z, codebook):
    B, D, H, W = z.shape
    HW = H * W
    K = codebook.shape[0]
    z3 = z.reshape(B, D, HW)
    T = HW
    x, idx = pl.pallas_call(
        _vq_body,
        grid=(B // BB,),
        in_specs=[
            pl.BlockSpec((BB, D, T), lambda b: (b, 0, 0)),
            pl.BlockSpec((K, D), lambda b: (0, 0)),
        ],
        out_specs=[
            pl.BlockSpec((BB, T, D), lambda b: (b, 0, 0)),
            pl.BlockSpec((BB, 1, 1, T), lambda b: (b, 0, 0, 0)),
        ],
        out_shape=[
            jax.ShapeDtypeStruct((B, HW, D), jnp.float32),
            jax.ShapeDtypeStruct((B, 1, 1, HW), jnp.int32),
        ],
        scratch_shapes=[pltpu.VMEM((K, 1), jnp.float32)],
    )(z3, codebook)
    return x, idx.reshape(B, HW)
